# baseline (speedup 1.0000x reference)
import functools

import jax
import jax.numpy as jnp
from jax import lax
from jax.experimental import pallas as pl
from jax.experimental.pallas import tpu as pltpu
from jax.experimental.pallas import tpu_sc as plsc

D_MODEL = 1024
SEQ = 8192
NUM_CORES = 2
NUM_SUBCORES = 16
NUM_WORKERS = NUM_CORES * NUM_SUBCORES

K_CHUNKS = 4
CHUNK_SEQ = SEQ // K_CHUNKS
ROWS_PER_WORKER = CHUNK_SEQ // NUM_WORKERS
S_BLK = 256
BLKS_PER_CHUNK = CHUNK_SEQ // S_BLK

_SC_MESH = plsc.VectorSubcoreMesh(core_axis_name="c", subcore_axis_name="s")


@functools.partial(
    pl.kernel,
    mesh=_SC_MESH,
    out_type=jax.ShapeDtypeStruct((CHUNK_SEQ, D_MODEL), jnp.float32),
    scratch_types=[
        pltpu.VMEM((ROWS_PER_WORKER,), jnp.int32),
        pltpu.VMEM((ROWS_PER_WORKER, D_MODEL), jnp.float32),
        pltpu.SemaphoreType.DMA,
    ],
)
def _gather_chunk_sc(table_hbm, idx_hbm, out_hbm, idx_v, rows_v, sem):
    wid = lax.axis_index("s") * NUM_CORES + lax.axis_index("c")
    base = wid * ROWS_PER_WORKER
    pltpu.sync_copy(idx_hbm.at[pl.ds(base, ROWS_PER_WORKER)], idx_v)
    pltpu.async_copy(table_hbm.at[idx_v], rows_v, sem).wait()
    pltpu.sync_copy(rows_v, out_hbm.at[pl.ds(base, ROWS_PER_WORKER)])


def _add_body(x_ref, p_ref, o_ref):
    o_ref[...] = x_ref[...] + p_ref[...][None, :, :]


def _add_body_carry(c_ref, x_ref, p_ref, o_ref):
    del c_ref
    o_ref[...] = x_ref[...] + p_ref[...][None, :, :]


def _add_chunk_tc(chunk_idx, carry, x, pos):
    b = x.shape[0]
    base_blk = chunk_idx * BLKS_PER_CHUNK

    def xmap(j, base_blk=base_blk):
        return (0, base_blk + j, 0)

    x_spec = pl.BlockSpec((b, S_BLK, D_MODEL), xmap)
    p_spec = pl.BlockSpec((S_BLK, D_MODEL), lambda j: (j, 0))
    o_spec = pl.BlockSpec((b, S_BLK, D_MODEL), xmap)
    out_shape = jax.ShapeDtypeStruct(x.shape, x.dtype)
    if carry is None:
        return pl.pallas_call(
            _add_body,
            grid=(BLKS_PER_CHUNK,),
            in_specs=[x_spec, p_spec],
            out_specs=o_spec,
            out_shape=out_shape,
        )(x, pos)
    return pl.pallas_call(
        _add_body_carry,
        grid=(BLKS_PER_CHUNK,),
        in_specs=[pl.BlockSpec(memory_space=pl.ANY), x_spec, p_spec],
        out_specs=o_spec,
        out_shape=out_shape,
        input_output_aliases={0: 0},
    )(carry, x, pos)


def kernel(x, region_ids, pos_table):
    ids = region_ids.astype(jnp.int32)
    pos_chunks = [
        _gather_chunk_sc(pos_table, ids[i * CHUNK_SEQ:(i + 1) * CHUNK_SEQ])
        for i in range(K_CHUNKS)
    ]
    carry = None
    for i in range(K_CHUNKS):
        carry = _add_chunk_tc(i, carry, x, pos_chunks[i])
    return carry


# speedup vs baseline: 1.0017x; 1.0017x over previous
import functools

import jax
import jax.numpy as jnp
from jax import lax
from jax.experimental import pallas as pl
from jax.experimental.pallas import tpu as pltpu
from jax.experimental.pallas import tpu_sc as plsc

D_MODEL = 1024
SEQ = 8192
NUM_CORES = 2
NUM_SUBCORES = 16
NUM_WORKERS = NUM_CORES * NUM_SUBCORES

K_CHUNKS = 4
CHUNK_SEQ = SEQ // K_CHUNKS
ROWS_PER_WORKER = CHUNK_SEQ // NUM_WORKERS
S_BLK = 512
BLKS_PER_CHUNK = CHUNK_SEQ // S_BLK

_SC_MESH = plsc.VectorSubcoreMesh(core_axis_name="c", subcore_axis_name="s")


@functools.partial(
    pl.kernel,
    mesh=_SC_MESH,
    out_type=jax.ShapeDtypeStruct((CHUNK_SEQ, D_MODEL), jnp.float32),
    scratch_types=[
        pltpu.VMEM((ROWS_PER_WORKER,), jnp.int32),
        pltpu.VMEM((ROWS_PER_WORKER, D_MODEL), jnp.float32),
        pltpu.SemaphoreType.DMA,
    ],
)
def _gather_chunk_sc(table_hbm, idx_hbm, out_hbm, idx_v, rows_v, sem):
    wid = lax.axis_index("s") * NUM_CORES + lax.axis_index("c")
    base = wid * ROWS_PER_WORKER
    pltpu.sync_copy(idx_hbm.at[pl.ds(base, ROWS_PER_WORKER)], idx_v)
    pltpu.async_copy(table_hbm.at[idx_v], rows_v, sem).wait()
    pltpu.sync_copy(rows_v, out_hbm.at[pl.ds(base, ROWS_PER_WORKER)])


def _add_body(x_ref, p_ref, o_ref):
    o_ref[...] = x_ref[...] + p_ref[...][None, :, :]


def _add_body_carry(c_ref, x_ref, p_ref, o_ref):
    del c_ref
    o_ref[...] = x_ref[...] + p_ref[...][None, :, :]


def _add_chunk_tc(chunk_idx, carry, x, pos):
    b = x.shape[0]
    base_blk = chunk_idx * BLKS_PER_CHUNK

    def xmap(j, base_blk=base_blk):
        return (0, base_blk + j, 0)

    x_spec = pl.BlockSpec((b, S_BLK, D_MODEL), xmap)
    p_spec = pl.BlockSpec((S_BLK, D_MODEL), lambda j: (j, 0))
    o_spec = pl.BlockSpec((b, S_BLK, D_MODEL), xmap)
    out_shape = jax.ShapeDtypeStruct(x.shape, x.dtype)
    if carry is None:
        return pl.pallas_call(
            _add_body,
            grid=(BLKS_PER_CHUNK,),
            in_specs=[x_spec, p_spec],
            out_specs=o_spec,
            out_shape=out_shape,
        )(x, pos)
    return pl.pallas_call(
        _add_body_carry,
        grid=(BLKS_PER_CHUNK,),
        in_specs=[pl.BlockSpec(memory_space=pl.ANY), x_spec, p_spec],
        out_specs=o_spec,
        out_shape=out_shape,
        input_output_aliases={0: 0},
    )(carry, x, pos)


def kernel(x, region_ids, pos_table):
    ids = region_ids.astype(jnp.int32)
    pos_chunks = [
        _gather_chunk_sc(pos_table, ids[i * CHUNK_SEQ:(i + 1) * CHUNK_SEQ])
        for i in range(K_CHUNKS)
    ]
    carry = None
    for i in range(K_CHUNKS):
        carry = _add_chunk_tc(i, carry, x, pos_chunks[i])
    return carry


# speedup vs baseline: 1.0019x; 1.0002x over previous
import functools

import jax
import jax.numpy as jnp
from jax import lax
from jax.experimental import pallas as pl
from jax.experimental.pallas import tpu as pltpu
from jax.experimental.pallas import tpu_sc as plsc

D_MODEL = 1024
SEQ = 8192
NUM_CORES = 2
NUM_SUBCORES = 16
NUM_WORKERS = NUM_CORES * NUM_SUBCORES

CHUNK0 = 2048
CHUNK1 = SEQ - CHUNK0
GATHER_INNER = 64
S_BLK = 512

_SC_MESH = plsc.VectorSubcoreMesh(core_axis_name="c", subcore_axis_name="s")


def _make_gather(seq_len):
    rows_per_worker = seq_len // NUM_WORKERS
    inner = min(GATHER_INNER, rows_per_worker)
    n_inner = rows_per_worker // inner

    @functools.partial(
        pl.kernel,
        mesh=_SC_MESH,
        out_type=jax.ShapeDtypeStruct((seq_len, D_MODEL), jnp.float32),
        scratch_types=[
            pltpu.VMEM((inner,), jnp.int32),
            pltpu.VMEM((inner, D_MODEL), jnp.float32),
            pltpu.SemaphoreType.DMA,
        ],
    )
    def _gather(table_hbm, idx_hbm, out_hbm, idx_v, rows_v, sem):
        wid = lax.axis_index("s") * NUM_CORES + lax.axis_index("c")
        base = wid * rows_per_worker

        def body(c, carry):
            off = base + c * inner
            pltpu.sync_copy(idx_hbm.at[pl.ds(off, inner)], idx_v)
            pltpu.async_copy(table_hbm.at[idx_v], rows_v, sem).wait()
            pltpu.sync_copy(rows_v, out_hbm.at[pl.ds(off, inner)])
            return carry

        lax.fori_loop(0, n_inner, body, 0)

    return _gather


_gather_small = _make_gather(CHUNK0)
_gather_large = _make_gather(CHUNK1)


def _add_body(x_ref, p_ref, o_ref):
    o_ref[...] = x_ref[...] + p_ref[...][None, :, :]


def _add_body_carry(c_ref, x_ref, p_ref, o_ref):
    del c_ref
    o_ref[...] = x_ref[...] + p_ref[...][None, :, :]


def _add_chunk_tc(seq_base, seq_len, carry, x, pos):
    b = x.shape[0]
    base_blk = seq_base // S_BLK
    nblk = seq_len // S_BLK

    def xmap(j, base_blk=base_blk):
        return (0, base_blk + j, 0)

    x_spec = pl.BlockSpec((b, S_BLK, D_MODEL), xmap)
    p_spec = pl.BlockSpec((S_BLK, D_MODEL), lambda j: (j, 0))
    o_spec = pl.BlockSpec((b, S_BLK, D_MODEL), xmap)
    out_shape = jax.ShapeDtypeStruct(x.shape, x.dtype)
    if carry is None:
        return pl.pallas_call(
            _add_body,
            grid=(nblk,),
            in_specs=[x_spec, p_spec],
            out_specs=o_spec,
            out_shape=out_shape,
        )(x, pos)
    return pl.pallas_call(
        _add_body_carry,
        grid=(nblk,),
        in_specs=[pl.BlockSpec(memory_space=pl.ANY), x_spec, p_spec],
        out_specs=o_spec,
        out_shape=out_shape,
        input_output_aliases={0: 0},
    )(carry, x, pos)


def kernel(x, region_ids, pos_table):
    ids = region_ids.astype(jnp.int32)
    pos0 = _gather_small(pos_table, ids[:CHUNK0])
    pos1 = _gather_large(pos_table, ids[CHUNK0:])
    out = _add_chunk_tc(0, CHUNK0, None, x, pos0)
    out = _add_chunk_tc(CHUNK0, CHUNK1, out, x, pos1)
    return out


# speedup vs baseline: 1.0172x; 1.0153x over previous
import functools

import jax
import jax.numpy as jnp
from jax import lax
from jax.experimental import pallas as pl
from jax.experimental.pallas import tpu as pltpu
from jax.experimental.pallas import tpu_sc as plsc

D_MODEL = 1024
SEQ = 8192
NUM_CORES = 2
NUM_SUBCORES = 16
NUM_WORKERS = NUM_CORES * NUM_SUBCORES

CHUNK0 = 2048
CHUNK1 = SEQ - CHUNK0
GATHER_INNER = 48
S_BLK = 512

_SC_MESH = plsc.VectorSubcoreMesh(core_axis_name="c", subcore_axis_name="s")


def _make_gather(seq_len):
    rows_per_worker = seq_len // NUM_WORKERS
    inner = min(GATHER_INNER, rows_per_worker // 2)
    n_inner = rows_per_worker // inner

    @functools.partial(
        pl.kernel,
        mesh=_SC_MESH,
        out_type=jax.ShapeDtypeStruct((seq_len, D_MODEL), jnp.float32),
        scratch_types=[
            pltpu.VMEM((rows_per_worker,), jnp.int32),
            pltpu.VMEM((inner, D_MODEL), jnp.float32),
            pltpu.VMEM((inner, D_MODEL), jnp.float32),
            pltpu.SemaphoreType.DMA,
            pltpu.SemaphoreType.DMA,
            pltpu.SemaphoreType.DMA,
        ],
    )
    def _gather(table_hbm, idx_hbm, out_hbm, idx_v, rows_a, rows_b,
                gsem, wsem_a, wsem_b):
        wid = lax.axis_index("s") * NUM_CORES + lax.axis_index("c")
        base = wid * rows_per_worker
        pltpu.sync_copy(idx_hbm.at[pl.ds(base, rows_per_worker)], idx_v)
        bufs = (rows_a, rows_b)
        wsems = (wsem_a, wsem_b)
        writes = [None, None]
        for c in range(n_inner):
            bi = c % 2
            if writes[bi] is not None:
                writes[bi].wait()
            pltpu.async_copy(
                table_hbm.at[idx_v.at[pl.ds(c * inner, inner)]],
                bufs[bi], gsem).wait()
            writes[bi] = pltpu.async_copy(
                bufs[bi], out_hbm.at[pl.ds(base + c * inner, inner)],
                wsems[bi])
        for w in writes:
            if w is not None:
                w.wait()

    return _gather


_gather_small = _make_gather(CHUNK0)
_gather_large = _make_gather(CHUNK1)


def _add_body(x_ref, p_ref, o_ref):
    o_ref[...] = x_ref[...] + p_ref[...][None, :, :]


def _add_body_carry(c_ref, x_ref, p_ref, o_ref):
    del c_ref
    o_ref[...] = x_ref[...] + p_ref[...][None, :, :]


def _add_chunk_tc(seq_base, seq_len, carry, x, pos):
    b = x.shape[0]
    base_blk = seq_base // S_BLK
    nblk = seq_len // S_BLK

    def xmap(j, base_blk=base_blk):
        return (0, base_blk + j, 0)

    x_spec = pl.BlockSpec((b, S_BLK, D_MODEL), xmap)
    p_spec = pl.BlockSpec((S_BLK, D_MODEL), lambda j: (j, 0))
    o_spec = pl.BlockSpec((b, S_BLK, D_MODEL), xmap)
    out_shape = jax.ShapeDtypeStruct(x.shape, x.dtype)
    if carry is None:
        return pl.pallas_call(
            _add_body,
            grid=(nblk,),
            in_specs=[x_spec, p_spec],
            out_specs=o_spec,
            out_shape=out_shape,
        )(x, pos)
    return pl.pallas_call(
        _add_body_carry,
        grid=(nblk,),
        in_specs=[pl.BlockSpec(memory_space=pl.ANY), x_spec, p_spec],
        out_specs=o_spec,
        out_shape=out_shape,
        input_output_aliases={0: 0},
    )(carry, x, pos)


def kernel(x, region_ids, pos_table):
    ids = region_ids.astype(jnp.int32)
    pos0 = _gather_small(pos_table, ids[:CHUNK0])
    pos1 = _gather_large(pos_table, ids[CHUNK0:])
    out = _add_chunk_tc(0, CHUNK0, None, x, pos0)
    out = _add_chunk_tc(CHUNK0, CHUNK1, out, x, pos1)
    return out
